# Initial kernel scaffold; baseline (speedup 1.0000x reference)
import jax
import jax.numpy as jnp
from jax.experimental import pallas as pl


def kernel(link_state, states_first, states_second, states_graph_ids, sates_num_edges, Wm1, bm1, Wm2, bm2, gru_k, gru_rk, gru_b, Wr1, br1, Wr2, br2):
    raise NotImplementedError("write your pallas kernel here")


# speedup vs baseline: 3.2825x; 3.2825x over previous
import functools

import jax
import jax.numpy as jnp
from jax import lax
from jax.experimental import pallas as pl
from jax.experimental.pallas import tpu as pltpu
from jax.experimental.pallas import tpu_sc as plsc

_NC = 2
_NS = 16
_NW = _NC * _NS
_K = 128
_T = 8
_G = 256


def _ab_body(h_ref, w1a_ref, w1b_ref, bm1_ref, a_ref, b_ref):
    h = h_ref[...]
    a_ref[...] = jnp.dot(h, w1a_ref[...], preferred_element_type=jnp.float32)
    b_ref[...] = (jnp.dot(h, w1b_ref[...], preferred_element_type=jnp.float32)
                  + bm1_ref[...])


def _mid_body(ag_ref, bg_ref, w2_ref, bm2_ref, m_ref):
    u = jnp.maximum(ag_ref[...] + bg_ref[...], 0.0)
    m_ref[...] = jnp.maximum(
        jnp.dot(u, w2_ref[...], preferred_element_type=jnp.float32)
        + bm2_ref[...], 0.0)


def _gru_ab_body(s2_ref, h_ref, gk_ref, grk_ref, gb0_ref, gb1_ref,
                 w1a_ref, w1b_ref, bm1_ref, ho_ref, a_ref, b_ref):
    x = s2_ref[0] + s2_ref[1]
    h = h_ref[...]
    mx = jnp.dot(x, gk_ref[...], preferred_element_type=jnp.float32) + gb0_ref[...]
    mh = jnp.dot(h, grk_ref[...], preferred_element_type=jnp.float32) + gb1_ref[...]
    d = h.shape[1]
    xz, xr, xh = mx[:, :d], mx[:, d:2 * d], mx[:, 2 * d:]
    rz, rr, rh = mh[:, :d], mh[:, d:2 * d], mh[:, 2 * d:]
    z = jax.nn.sigmoid(xz + rz)
    r = jax.nn.sigmoid(xr + rr)
    hh = jnp.tanh(xh + r * rh)
    hn = z * h + (1.0 - z) * hh
    ho_ref[...] = hn
    a_ref[...] = jnp.dot(hn, w1a_ref[...], preferred_element_type=jnp.float32)
    b_ref[...] = (jnp.dot(hn, w1b_ref[...], preferred_element_type=jnp.float32)
                  + bm1_ref[...])


def _pool_body(ids_ref, h_ref, wr1_ref, br1_ref, wr2t_ref, br2_ref,
               out_ref, acc_ref):
    i = pl.program_id(0)

    @pl.when(i == 0)
    def _():
        acc_ref[...] = jnp.zeros_like(acc_ref)

    ids = ids_ref[0]
    bn = ids.shape[1]
    oh = (lax.broadcasted_iota(jnp.int32, (_G, bn), 0)
          == jnp.broadcast_to(ids, (_G, bn))).astype(jnp.float32)
    acc_ref[...] += jnp.dot(oh, h_ref[...], preferred_element_type=jnp.float32)

    @pl.when(i == pl.num_programs(0) - 1)
    def _():
        p = acc_ref[...]
        r = jnp.maximum(
            jnp.dot(p, wr1_ref[...], preferred_element_type=jnp.float32)
            + br1_ref[...], 0.0)
        out_ref[...] = (jnp.sum(r * wr2t_ref[...], axis=1, keepdims=True)
                        + br2_ref[...])


def _sc_gather_body(a_hbm, b_hbm, f_hbm, s_hbm, ag_hbm, bg_hbm,
                    i1, i2, r1, r2, sem1, sem2):
    cid = lax.axis_index("c")
    sid = lax.axis_index("s")
    wid = sid * _NC + cid
    e = f_hbm.shape[0]
    nch = e // _K

    def body(i, carry):
        c = wid + i * _NW

        @pl.when(c < nch)
        def _():
            base = c * _K
            pltpu.sync_copy(f_hbm.at[pl.ds(base, _K)], i1)
            pltpu.sync_copy(s_hbm.at[pl.ds(base, _K)], i2)
            d1 = pltpu.async_copy(a_hbm.at[i1], r1, sem1)
            d2 = pltpu.async_copy(b_hbm.at[i2], r2, sem2)
            d1.wait()
            d2.wait()
            pltpu.sync_copy(r1, ag_hbm.at[pl.ds(base, _K), :])
            pltpu.sync_copy(r2, bg_hbm.at[pl.ds(base, _K), :])

        return carry

    lax.fori_loop(0, pl.cdiv(nch, _NW), body, 0)


def _sc_scatter_body(m_hbm, s_hbm, z_hbm, out_hbm, idx_v, rows_v, acc):
    cid = lax.axis_index("c")
    sid = lax.axis_index("s")
    n_pad = acc.shape[0]
    rows_pt = n_pad // _NS
    e = s_hbm.shape[0]
    nch_half = (e // _K) // _NC

    pltpu.sync_copy(z_hbm, acc.at[pl.ds(sid * rows_pt, rows_pt), :])
    plsc.subcore_barrier()

    def body(i, carry):
        ch = sid + _NS * i

        @pl.when(ch < nch_half)
        def _():
            base = (ch * _NC + cid) * _K
            pltpu.sync_copy(s_hbm.at[pl.ds(base, _K)], idx_v)
            pltpu.sync_copy(m_hbm.at[pl.ds(base, _K), :], rows_v)
            pltpu.sync_copy(rows_v, acc.at[idx_v], add=True)

        return carry

    lax.fori_loop(0, pl.cdiv(nch_half, _NS), body, 0)
    plsc.subcore_barrier()
    pltpu.sync_copy(acc.at[pl.ds(sid * rows_pt, rows_pt), :],
                    out_hbm.at[cid, pl.ds(sid * rows_pt, rows_pt), :])


def kernel(link_state, states_first, states_second, states_graph_ids,
           sates_num_edges, Wm1, bm1, Wm2, bm2, gru_k, gru_rk, gru_b,
           Wr1, br1, Wr2, br2):
    n, d = link_state.shape
    e = states_first.shape[0]
    f32 = jnp.float32

    w1a = Wm1[:d]
    w1b = Wm1[d:]
    bm1r = bm1.reshape(1, d)
    bm2r = bm2.reshape(1, d)
    gb0 = gru_b[0:1]
    gb1 = gru_b[1:2]
    br1r = br1.reshape(1, d)
    wr2t = Wr2.reshape(1, d)
    br2r = br2.reshape(1, 1)
    n_pad = ((n + 8 * _NS - 1) // (8 * _NS)) * (8 * _NS)
    zrows = jnp.zeros((n_pad // _NS, d), f32)

    bn = 2000
    nb = n // bn
    be = 2560
    neb = e // be

    wspec = pl.BlockSpec((d, d), lambda i: (0, 0))
    w3spec = pl.BlockSpec((d, 3 * d), lambda i: (0, 0))
    bspec = pl.BlockSpec((1, d), lambda i: (0, 0))
    b3spec = pl.BlockSpec((1, 3 * d), lambda i: (0, 0))

    ab_call = pl.pallas_call(
        _ab_body,
        grid=(nb,),
        in_specs=[pl.BlockSpec((bn, d), lambda i: (i, 0)), wspec, wspec, bspec],
        out_specs=[pl.BlockSpec((bn, d), lambda i: (i, 0))] * 2,
        out_shape=[jax.ShapeDtypeStruct((n, d), f32)] * 2,
    )

    mid_call = pl.pallas_call(
        _mid_body,
        grid=(neb,),
        in_specs=[pl.BlockSpec((be, d), lambda i: (i, 0)),
                  pl.BlockSpec((be, d), lambda i: (i, 0)), wspec, bspec],
        out_specs=pl.BlockSpec((be, d), lambda i: (i, 0)),
        out_shape=jax.ShapeDtypeStruct((e, d), f32),
    )

    gru_ab_call = pl.pallas_call(
        _gru_ab_body,
        grid=(nb,),
        in_specs=[pl.BlockSpec((_NC, bn, d), lambda i: (0, i, 0)),
                  pl.BlockSpec((bn, d), lambda i: (i, 0)),
                  w3spec, w3spec, b3spec, b3spec, wspec, wspec, bspec],
        out_specs=[pl.BlockSpec((bn, d), lambda i: (i, 0))] * 3,
        out_shape=[jax.ShapeDtypeStruct((n, d), f32)] * 3,
    )

    pool_call = pl.pallas_call(
        _pool_body,
        grid=(nb,),
        in_specs=[pl.BlockSpec((1, 1, bn), lambda i: (i, 0, 0)),
                  pl.BlockSpec((bn, d), lambda i: (i, 0)),
                  wspec, bspec, bspec, pl.BlockSpec((1, 1), lambda i: (0, 0))],
        out_specs=pl.BlockSpec((_G, 1), lambda i: (0, 0)),
        out_shape=jax.ShapeDtypeStruct((_G, 1), f32),
        scratch_shapes=[pltpu.VMEM((_G, d), f32)],
        compiler_params=pltpu.CompilerParams(
            dimension_semantics=("arbitrary",)),
    )

    mesh = plsc.VectorSubcoreMesh(core_axis_name="c", subcore_axis_name="s")

    gather_call = functools.partial(
        pl.kernel, mesh=mesh,
        out_type=[jax.ShapeDtypeStruct((e, d), f32)] * 2,
        scratch_types=[pltpu.VMEM((_K,), jnp.int32),
                       pltpu.VMEM((_K,), jnp.int32),
                       pltpu.VMEM((_K, d), f32),
                       pltpu.VMEM((_K, d), f32),
                       pltpu.SemaphoreType.DMA,
                       pltpu.SemaphoreType.DMA],
    )(_sc_gather_body)

    scatter_call = functools.partial(
        pl.kernel, mesh=mesh,
        out_type=jax.ShapeDtypeStruct((_NC, n_pad, d), f32),
        scratch_types=[pltpu.VMEM((_K,), jnp.int32),
                       pltpu.VMEM((_K, d), f32),
                       pltpu.VMEM_SHARED((n_pad, d), f32)],
    )(_sc_scatter_body)

    ids3 = states_graph_ids.astype(jnp.int32).reshape(nb, 1, bn)

    h = link_state
    a, b = ab_call(h, w1a, w1b, bm1r)
    for _ in range(_T):
        ag, bg = gather_call(a, b, states_first, states_second)
        m = mid_call(ag, bg, Wm2, bm2r)
        s2 = scatter_call(m, states_second, zrows)
        h, a, b = gru_ab_call(s2, h, gru_k, gru_rk, gb0, gb1, w1a, w1b, bm1r)
    out = pool_call(ids3, h, Wr1, br1r, wr2t, br2r)
    return out
